# baseline (speedup 1.0000x reference)
import functools

import jax
import jax.numpy as jnp
from jax import lax
from jax.experimental import pallas as pl
from jax.experimental.pallas import tpu as pltpu
from jax.experimental.pallas import tpu_sc as plsc

NW = 32
CH = 128
K = 8
S = CH * K


def _gather_call(n, d):
    per_w = n // NW
    nch = per_w // S
    mesh = plsc.VectorSubcoreMesh(core_axis_name="c", subcore_axis_name="s")

    @functools.partial(
        pl.kernel,
        mesh=mesh,
        out_type=jax.ShapeDtypeStruct((n, d), jnp.float32),
        scratch_types=[
            pltpu.VMEM((K, CH), jnp.int32),
            pltpu.VMEM((S, d), jnp.float32),
            pltpu.SemaphoreType.DMA,
        ],
        compiler_params=pltpu.CompilerParams(use_tc_tiling_on_sc=False),
    )
    def k(idx_hbm, table_hbm, out_hbm, idx_v, rows_v, sem):
        wid = lax.axis_index("s") * 2 + lax.axis_index("c")
        wbase = wid * nch

        def body(g, carry):
            row = wbase + g
            pltpu.sync_copy(idx_hbm.at[row], idx_v)
            cps = [
                pltpu.async_copy(
                    table_hbm.at[idx_v.at[j]],
                    rows_v.at[pl.ds(j * CH, CH)],
                    sem,
                )
                for j in range(K)
            ]
            for cp in cps:
                cp.wait()
            pltpu.sync_copy(rows_v, out_hbm.at[pl.ds(row * S, S)])
            return carry

        lax.fori_loop(0, nch, body, 0)

    return k


def kernel(x, table):
    n = x.size
    d = table.shape[1]
    idx = x.astype(jnp.int32).reshape(n // (K * CH), K, CH)
    out = _gather_call(n, d)(idx, table)
    return out.reshape(x.shape + (d,))


# speedup vs baseline: 1.0188x; 1.0188x over previous
import functools

import jax
import jax.numpy as jnp
from jax import lax
from jax.experimental import pallas as pl
from jax.experimental.pallas import tpu as pltpu
from jax.experimental.pallas import tpu_sc as plsc

NW = 32
CH = 128
K = 4
S = CH * K


def _gather_call(n, d):
    per_w = n // NW
    nch = per_w // S
    nidx = per_w // CH
    mesh = plsc.VectorSubcoreMesh(core_axis_name="c", subcore_axis_name="s")

    @functools.partial(
        pl.kernel,
        mesh=mesh,
        out_type=jax.ShapeDtypeStruct((n, d), jnp.float32),
        scratch_types=[
            pltpu.VMEM((nidx, CH), jnp.int32),
            pltpu.VMEM((S, d), jnp.float32),
            pltpu.VMEM((S, d), jnp.float32),
            pltpu.SemaphoreType.DMA,
            pltpu.SemaphoreType.DMA,
            pltpu.SemaphoreType.DMA,
            pltpu.SemaphoreType.DMA,
        ],
        compiler_params=pltpu.CompilerParams(use_tc_tiling_on_sc=False),
    )
    def k(idx_hbm, table_hbm, out_hbm, idx_v, rows0, rows1, sg0, sg1, so0, so1):
        wid = lax.axis_index("s") * 2 + lax.axis_index("c")
        obase = wid * per_w
        rows = (rows0, rows1)
        sg = (sg0, sg1)
        so = (so0, so1)

        pltpu.sync_copy(idx_hbm.at[wid], idx_v)

        def fire_gathers(s, g):
            for j in range(K):
                pltpu.async_copy(
                    table_hbm.at[idx_v.at[g * K + j]],
                    rows[s].at[pl.ds(j * CH, CH)],
                    sg[s],
                )

        def wait_gathers(s):
            pltpu.make_async_copy(
                out_hbm.at[pl.ds(0, S)], rows[s], sg[s]
            ).wait()

        def fire_out(s, g):
            pltpu.async_copy(rows[s], out_hbm.at[pl.ds(obase + g * S, S)], so[s])

        def wait_out(s):
            pltpu.make_async_copy(
                rows[s], out_hbm.at[pl.ds(obase, S)], so[s]
            ).wait()

        fire_gathers(0, 0)
        fire_gathers(1, 1)
        wait_gathers(0)
        fire_out(0, 0)

        def step(g, s):
            s1 = 1 - s
            wait_out(s1)
            fire_gathers(s1, g + 1)
            wait_gathers(s)
            fire_out(s, g)

        def pair_body(t, carry):
            step(1 + 2 * t, 1)
            step(2 + 2 * t, 0)
            return carry

        lax.fori_loop(0, (nch - 2) // 2, pair_body, 0)

        wait_gathers(1)
        fire_out(1, nch - 1)
        wait_out(0)
        wait_out(1)

    return k


def kernel(x, table):
    n = x.size
    d = table.shape[1]
    idx = x.astype(jnp.int32).reshape(NW, n // (NW * CH), CH)
    out = _gather_call(n, d)(idx, table)
    return out.reshape(x.shape + (d,))
